# baseline (speedup 1.0000x reference)
import jax
import jax.numpy as jnp
from jax import lax
from jax.experimental import pallas as pl
from jax.experimental.pallas import tpu as pltpu
from jax.experimental.pallas import tpu_sc as plsc

B = 16384
D = 32
NC, NS, L = 2, 16, 16
NW = NC * NS
BPW = B // NW
NP = 2
BPP = BPW // NP
NGP = BPP // L


def _mf_body(u_hbm, i_hbm, ub_hbm, uv_hbm, ib_hbm, iv_hbm, g_hbm, lane_hbm,
             out_hbm,
             u_idx, i_idx, u4, i4, vu, vi, bu, bi, outv, gv, lanev, tbuf,
             sem):
    wid = lax.axis_index("s") * NC + lax.axis_index("c")

    pltpu.sync_copy(u_hbm.at[wid], u_idx)
    pltpu.sync_copy(i_hbm.at[wid], i_idx)
    pltpu.sync_copy(g_hbm, gv)
    pltpu.sync_copy(lane_hbm, lanev)

    bias_copies = [
        pltpu.async_copy(ub_hbm.at[u_idx], bu, sem),
        pltpu.async_copy(ib_hbm.at[i_idx], bi, sem),
    ]

    gvv = gv[...]
    lane16 = lanev[...]

    for p in range(NP):
        poff = p * BPP
        def mkidx(c, _):
            base = pl.multiple_of(c * L, L)
            s = pl.ds(base, L)
            sp = pl.ds(poff + base, L)
            u4[s] = lax.shift_right_logical(u_idx[sp], 2)
            i4[s] = lax.shift_right_logical(i_idx[sp], 2)
            return _
        lax.fori_loop(0, NGP, mkidx, 0)

        copies = [
            pltpu.async_copy(uv_hbm.at[u4], vu, sem),
            pltpu.async_copy(iv_hbm.at[i4], vi, sem),
        ]
        for cp in copies:
            cp.wait()

        def group(gg, _):
            gbase = pl.multiple_of(gg * L, L)
            uvals = u_idx[pl.ds(poff + gbase, L)]
            ivals = i_idx[pl.ds(poff + gbase, L)]
            for r in range(L):
                b = gbase + r
                ou = (uvals[r] & 3) * 32
                oi = (ivals[r] & 3) * 32
                a0 = vu[b, pl.ds(ou, L)]
                a1 = vu[b, pl.ds(ou + L, L)]
                c0 = vi[b, pl.ds(oi, L)]
                c1 = vi[b, pl.ds(oi + L, L)]
                plsc.store_scatter(tbuf, [lane16 + r], a0 * c0 + a1 * c1)
            acc = tbuf[pl.ds(0, L)]
            for l in range(1, L):
                acc = acc + tbuf[pl.ds(l * L, L)]
            outv[pl.ds(poff + gbase, L)] = acc
            return _

        lax.fori_loop(0, NGP, group, 0)

    for cp in bias_copies:
        cp.wait()

    def addbias(c, _):
        s = pl.ds(pl.multiple_of(c * L, L), L)
        outv[s] = outv[s] + gvv + bu[s] + bi[s]
        return _
    lax.fori_loop(0, BPW // L, addbias, 0)

    pltpu.sync_copy(outv, out_hbm.at[pl.ds(wid * BPW, BPW)])


@jax.jit
def _mf(u, i, user_bias, user_vec, item_bias, item_vec, glob_bias, lane):
    mesh = plsc.VectorSubcoreMesh(core_axis_name="c", subcore_axis_name="s",
                                  num_cores=NC, num_subcores=NS)
    return pl.kernel(
        _mf_body,
        out_type=jax.ShapeDtypeStruct((B,), jnp.float32),
        mesh=mesh,
        compiler_params=pltpu.CompilerParams(
            needs_layout_passes=False, use_tc_tiling_on_sc=True),
        scratch_types=[
            pltpu.VMEM((BPW,), jnp.int32),
            pltpu.VMEM((BPW,), jnp.int32),
            pltpu.VMEM((BPP,), jnp.int32),
            pltpu.VMEM((BPP,), jnp.int32),
            pltpu.VMEM((BPP, 128), jnp.float32),
            pltpu.VMEM((BPP, 128), jnp.float32),
            pltpu.VMEM((BPW,), jnp.float32),
            pltpu.VMEM((BPW,), jnp.float32),
            pltpu.VMEM((BPW,), jnp.float32),
            pltpu.VMEM((L,), jnp.float32),
            pltpu.VMEM((L,), jnp.int32),
            pltpu.VMEM((L * L,), jnp.float32),
            pltpu.SemaphoreType.DMA,
        ],
    )(u, i, user_bias, user_vec.reshape(-1, 128), item_bias,
      item_vec.reshape(-1, 128), glob_bias, lane)


def kernel(u, i, user_bias, user_vec, item_bias, item_vec, glob_bias):
    u = u.astype(jnp.int32).reshape(NW, BPW)
    i = i.astype(jnp.int32).reshape(NW, BPW)
    glob = jnp.broadcast_to(glob_bias.reshape(1), (L,))
    lane = (jnp.arange(L, dtype=jnp.int32) * L)
    return _mf(u, i, user_bias, user_vec, item_bias, item_vec, glob, lane)
